# baseline (speedup 1.0000x reference)
import functools

import jax
import jax.numpy as jnp
from jax import lax
from jax.experimental import pallas as pl
from jax.experimental.pallas import tpu as pltpu
from jax.experimental.pallas import tpu_sc as plsc

NUM_TAGS = 44
B, L = 16384, 108
OUT_W = L + 1
LANES = 16
NC, NS = 2, 16
NW = NC * NS
CHUNK = 128
TAB_PAD = 48
UNROLL = 2
N_SPLITS = 2

_ROW_OFFS = (0, 16, 32, 48, 64, 80, L - LANES)

_MESH = plsc.VectorSubcoreMesh(core_axis_name="c", subcore_axis_name="s")


@functools.cache
def _make_lookup(rows):
    rows_per_w = rows // NW
    n_chunks = rows_per_w // CHUNK

    @functools.partial(
        pl.kernel,
        out_type=jax.ShapeDtypeStruct((rows, OUT_W), jnp.int32),
        mesh=_MESH,
        compiler_params=pltpu.CompilerParams(needs_layout_passes=False),
        scratch_types=[
            pltpu.VMEM((TAB_PAD,), jnp.int32),
            pltpu.VMEM((CHUNK, L), jnp.int32),
            pltpu.VMEM((CHUNK, L), jnp.int32),
            pltpu.VMEM((CHUNK, OUT_W), jnp.int32),
            pltpu.VMEM((CHUNK, OUT_W), jnp.int32),
            pltpu.SemaphoreType.DMA,
            pltpu.SemaphoreType.DMA,
            pltpu.SemaphoreType.DMA,
            pltpu.SemaphoreType.DMA,
        ],
    )
    def _sc_lookup(tags_hbm, table_hbm, out_hbm,
                   tab_v, tags0, tags1, out0, out1,
                   in_sem0, in_sem1, out_sem0, out_sem1):
        wid = lax.axis_index("s") * NC + lax.axis_index("c")
        base = wid * rows_per_w
        tags_bufs, out_bufs = [tags0, tags1], [out0, out1]
        in_sems, out_sems = [in_sem0, in_sem1], [out_sem0, out_sem1]

        def start_in(ci):
            r0 = base + ci * CHUNK
            return pltpu.async_copy(tags_hbm.at[pl.ds(r0, CHUNK)],
                                    tags_bufs[ci % 2],
                                    in_sems[ci % 2])

        in_dma = [start_in(0), None]
        if n_chunks > 1:
            in_dma[1] = start_in(1)
        pltpu.sync_copy(table_hbm, tab_v)
        zero_vals = jnp.zeros((LANES,), jnp.int32)
        zero_cols = jnp.zeros((LANES,), jnp.int32)
        for i in range(CHUNK // LANES):
            rows16 = lax.iota(jnp.int32, LANES) + i * LANES
            plsc.store_scatter(out0, [rows16, zero_cols], zero_vals)
            plsc.store_scatter(out1, [rows16, zero_cols], zero_vals)

        out_dma = [None, None]
        for ci in range(n_chunks):
            b = ci % 2
            in_dma[b].wait()
            if out_dma[b] is not None:
                out_dma[b].wait()
            tbuf, obuf = tags_bufs[b], out_bufs[b]

            @plsc.parallel_loop(0, CHUNK, 1, unroll=UNROLL)
            def _gather_row(r):
                trow, orow = tbuf.at[r], obuf.at[r]
                for off in _ROW_OFFS:
                    tag16 = trow[pl.ds(off, LANES)]
                    lab16 = plsc.load_gather(tab_v, [tag16])
                    orow[pl.ds(off + 1, LANES)] = lab16

            if ci + 2 < n_chunks:
                in_dma[b] = start_in(ci + 2)
            r0 = base + ci * CHUNK
            out_dma[b] = pltpu.async_copy(
                obuf, out_hbm.at[pl.ds(r0, CHUNK)], out_sems[b])
        for b in range(min(2, n_chunks)):
            out_dma[b].wait()

    return _sc_lookup


def kernel(tags, label_table):
    table_pad = jnp.zeros((TAB_PAD,), jnp.int32).at[:NUM_TAGS].set(label_table)
    rows = B // N_SPLITS
    lookup = _make_lookup(rows)
    parts = [lookup(lax.slice_in_dim(tags, i * rows, (i + 1) * rows), table_pad)
             for i in range(N_SPLITS)]
    return jnp.concatenate(parts, axis=0) if len(parts) > 1 else parts[0]


# speedup vs baseline: 1.4187x; 1.4187x over previous
import functools

import jax
import jax.numpy as jnp
import numpy as np
from jax import lax
from jax.experimental import pallas as pl
from jax.experimental.pallas import tpu as pltpu
from jax.experimental.pallas import tpu_sc as plsc

NUM_TAGS = 44
B, L = 16384, 108
OUT_W = L + 1
LANES = 16
NC, NS = 2, 16
NW = NC * NS
ROWS_PER_W = B // NW
CHUNK = 128
N_CHUNKS = ROWS_PER_W // CHUNK
OUT_CH = CHUNK * OUT_W
SENT_ROW = CHUNK
TAB_PAD = 48
UNROLL = 4


_ROW_OFFS = (0, 16, 32, 48, 64, 80, L - LANES)

_MESH = plsc.VectorSubcoreMesh(core_axis_name="c", subcore_axis_name="s")


@functools.partial(
    pl.kernel,
    out_type=jax.ShapeDtypeStruct((B, OUT_W), jnp.int32),
    mesh=_MESH,
    compiler_params=pltpu.CompilerParams(needs_layout_passes=False),
    scratch_types=[
        pltpu.VMEM((TAB_PAD,), jnp.int32),
        pltpu.VMEM((CHUNK, L), jnp.int32),
        pltpu.VMEM((CHUNK, L), jnp.int32),
        pltpu.VMEM((CHUNK, OUT_W), jnp.int32),
        pltpu.VMEM((CHUNK, OUT_W), jnp.int32),
        pltpu.SemaphoreType.DMA,
        pltpu.SemaphoreType.DMA,
        pltpu.SemaphoreType.DMA,
        pltpu.SemaphoreType.DMA,
    ],
)
def _sc_lookup(tags_hbm, table_hbm, out_hbm,
               tab_v, tags0, tags1, out0, out1,
               in_sem0, in_sem1, out_sem0, out_sem1):
    wid = lax.axis_index("s") * NC + lax.axis_index("c")
    base = wid * ROWS_PER_W
    tags_bufs, out_bufs = [tags0, tags1], [out0, out1]
    in_sems, out_sems = [in_sem0, in_sem1], [out_sem0, out_sem1]

    def start_in(ci):
        r0 = base + ci * CHUNK
        return pltpu.async_copy(tags_hbm.at[pl.ds(r0, CHUNK)],
                                tags_bufs[ci % 2],
                                in_sems[ci % 2])

    in_dma = [start_in(0), None]
    if N_CHUNKS > 1:
        in_dma[1] = start_in(1)
    pltpu.sync_copy(table_hbm, tab_v)
    zero_vals = jnp.zeros((LANES,), jnp.int32)
    zero_cols = jnp.zeros((LANES,), jnp.int32)
    for i in range(CHUNK // LANES):
        rows16 = lax.iota(jnp.int32, LANES) + i * LANES
        plsc.store_scatter(out0, [rows16, zero_cols], zero_vals)
        plsc.store_scatter(out1, [rows16, zero_cols], zero_vals)

    out_dma = [None, None]
    for ci in range(N_CHUNKS):
        b = ci % 2
        in_dma[b].wait()
        if out_dma[b] is not None:
            out_dma[b].wait()
        tbuf, obuf = tags_bufs[b], out_bufs[b]

        @plsc.parallel_loop(0, CHUNK, 1, unroll=UNROLL)
        def _gather_row(r):
            trow, orow = tbuf.at[r], obuf.at[r]
            for off in _ROW_OFFS:
                tag16 = trow[pl.ds(off, LANES)]
                lab16 = plsc.load_gather(tab_v, [tag16])
                orow[pl.ds(off + 1, LANES)] = lab16

        if ci + 2 < N_CHUNKS:
            in_dma[b] = start_in(ci + 2)
        r0 = base + ci * CHUNK
        out_dma[b] = pltpu.async_copy(
            obuf, out_hbm.at[pl.ds(r0, CHUNK)], out_sems[b])
    for b in range(min(2, N_CHUNKS)):
        out_dma[b].wait()


def kernel(tags, label_table):
    table_pad = jnp.zeros((TAB_PAD,), jnp.int32).at[:NUM_TAGS].set(label_table)
    return _sc_lookup(tags, table_pad)


# speedup vs baseline: 1.5719x; 1.1080x over previous
import functools

import jax
import jax.numpy as jnp
from jax import lax
from jax.experimental import pallas as pl
from jax.experimental.pallas import tpu as pltpu
from jax.experimental.pallas import tpu_sc as plsc

NUM_TAGS = 44
B, L = 16384, 108
OUT_W = L + 1
LANES = 16
NC, NS = 2, 16
NW = NC * NS
COLS_PER_W = B // NW
BLK = 8
N_BLKS = (L + BLK - 1) // BLK
TAB_PAD = 48
UNROLL = 4
N_VEC = COLS_PER_W // LANES

_MESH = plsc.VectorSubcoreMesh(core_axis_name="c", subcore_axis_name="s")


@functools.partial(
    pl.kernel,
    out_type=jax.ShapeDtypeStruct((OUT_W, B), jnp.int32),
    mesh=_MESH,
    compiler_params=pltpu.CompilerParams(needs_layout_passes=False),
    scratch_types=[
        pltpu.VMEM((TAB_PAD,), jnp.int32),
        pltpu.VMEM((BLK, COLS_PER_W), jnp.int32),
        pltpu.VMEM((BLK, COLS_PER_W), jnp.int32),
        pltpu.VMEM((BLK, COLS_PER_W), jnp.int32),
        pltpu.VMEM((BLK, COLS_PER_W), jnp.int32),
        pltpu.VMEM((BLK, COLS_PER_W), jnp.int32),
        pltpu.SemaphoreType.DMA,
        pltpu.SemaphoreType.DMA,
        pltpu.SemaphoreType.DMA,
        pltpu.SemaphoreType.DMA,
        pltpu.SemaphoreType.DMA,
    ],
)
def _sc_lookup_t(tags_hbm, table_hbm, out_hbm,
                 tab_v, in0, in1, in2, out0, out1,
                 in_sem0, in_sem1, in_sem2, out_sem0, out_sem1):
    wid = lax.axis_index("s") * NC + lax.axis_index("c")
    c0 = wid * COLS_PER_W
    in_bufs, out_bufs = [in0, in1, in2], [out0, out1]
    in_sems, out_sems = [in_sem0, in_sem1, in_sem2], [out_sem0, out_sem1]

    def rows_of(tr):
        return min(BLK, L - tr * BLK)

    def start_in(tr):
        nr = rows_of(tr)
        return pltpu.async_copy(
            tags_hbm.at[pl.ds(tr * BLK, nr), pl.ds(c0, COLS_PER_W)],
            in_bufs[tr % 3].at[pl.ds(0, nr)],
            in_sems[tr % 3])

    in_dma = [start_in(0), start_in(1), None]
    pltpu.sync_copy(table_hbm, tab_v)

    def map_row(src_buf, srow, dst_buf, drow):
        @plsc.parallel_loop(0, N_VEC, 1, unroll=UNROLL)
        def _map_vec(i):
            tag16 = src_buf[srow, pl.ds(i * LANES, LANES)]
            dst_buf[drow, pl.ds(i * LANES, LANES)] = (
                plsc.load_gather(tab_v, [tag16]))

    def zero_row(dst_buf, drow):
        z16 = jnp.zeros((LANES,), jnp.int32)

        @plsc.parallel_loop(0, N_VEC, 1, unroll=UNROLL)
        def _zero_vec(i):
            dst_buf[drow, pl.ds(i * LANES, LANES)] = z16

    out_dma = [None, None]
    for tr in range(N_BLKS):
        bi, bo = tr % 3, tr % 2
        in_dma[bi].wait()
        if out_dma[bo] is not None:
            out_dma[bo].wait()
        cur, prev, obuf = in_bufs[bi], in_bufs[(tr - 1) % 3], out_bufs[bo]
        n_out = min(BLK, OUT_W - tr * BLK)
        if tr == 0:
            zero_row(obuf, 0)
        else:
            map_row(prev, BLK - 1, obuf, 0)
        for s in range(1, n_out):
            map_row(cur, s - 1, obuf, s)
        if tr + 2 < N_BLKS:
            in_dma[(tr + 2) % 3] = start_in(tr + 2)
        out_dma[bo] = pltpu.async_copy(
            obuf.at[pl.ds(0, n_out)],
            out_hbm.at[pl.ds(tr * BLK, n_out), pl.ds(c0, COLS_PER_W)],
            out_sems[bo])
    for b in range(2):
        out_dma[b].wait()


def kernel(tags, label_table):
    table_pad = jnp.zeros((TAB_PAD,), jnp.int32).at[:NUM_TAGS].set(label_table)
    out_t = _sc_lookup_t(tags.T, table_pad)
    return out_t.T


# speedup vs baseline: 1.8899x; 1.2023x over previous
import functools

import jax
import jax.numpy as jnp
from jax import lax
from jax.experimental import pallas as pl
from jax.experimental.pallas import tpu as pltpu
from jax.experimental.pallas import tpu_sc as plsc

NUM_TAGS = 44
B, L = 16384, 108
OUT_W = L + 1
LANES = 16
NC, NS = 2, 16
NW = NC * NS
COLS_PER_W = B // NW
BLK = 8
N_BLKS = (L + BLK - 1) // BLK
TAB_PAD = 48
UNROLL = 4
N_VEC = COLS_PER_W // LANES

_MESH = plsc.VectorSubcoreMesh(core_axis_name="c", subcore_axis_name="s")


@functools.partial(
    pl.kernel,
    out_type=jax.ShapeDtypeStruct((OUT_W, B), jnp.int32),
    mesh=_MESH,
    compiler_params=pltpu.CompilerParams(needs_layout_passes=False),
    scratch_types=[
        pltpu.VMEM((TAB_PAD,), jnp.int32),
        pltpu.VMEM((BLK, COLS_PER_W), jnp.int32),
        pltpu.VMEM((BLK, COLS_PER_W), jnp.int32),
        pltpu.VMEM((BLK, COLS_PER_W), jnp.int32),
        pltpu.VMEM((BLK, COLS_PER_W), jnp.int32),
        pltpu.VMEM((BLK, COLS_PER_W), jnp.int32),
        pltpu.SemaphoreType.DMA,
        pltpu.SemaphoreType.DMA,
        pltpu.SemaphoreType.DMA,
        pltpu.SemaphoreType.DMA,
        pltpu.SemaphoreType.DMA,
    ],
)
def _sc_lookup_t(tags_hbm, table_hbm, out_hbm,
                 tab_v, in0, in1, in2, out0, out1,
                 in_sem0, in_sem1, in_sem2, out_sem0, out_sem1):
    wid = lax.axis_index("s") * NC + lax.axis_index("c")
    c0 = wid * COLS_PER_W
    in_bufs, out_bufs = [in0, in1, in2], [out0, out1]
    in_sems, out_sems = [in_sem0, in_sem1, in_sem2], [out_sem0, out_sem1]

    def rows_of(tr):
        return min(BLK, L - tr * BLK)

    def start_in(tr):
        nr = rows_of(tr)
        return pltpu.async_copy(
            tags_hbm.at[pl.ds(tr * BLK, nr), pl.ds(c0, COLS_PER_W)],
            in_bufs[tr % 3].at[pl.ds(0, nr)],
            in_sems[tr % 3])

    in_dma = [start_in(0), start_in(1), None]
    pltpu.sync_copy(table_hbm, tab_v)

    def map_row(src_buf, srow, dst_buf, drow):
        @plsc.parallel_loop(0, N_VEC, 1, unroll=UNROLL)
        def _map_vec(i):
            tag16 = src_buf[srow, pl.ds(i * LANES, LANES)]
            dst_buf[drow, pl.ds(i * LANES, LANES)] = (
                plsc.load_gather(tab_v, [tag16]))

    def zero_row(dst_buf, drow):
        z16 = jnp.zeros((LANES,), jnp.int32)

        @plsc.parallel_loop(0, N_VEC, 1, unroll=UNROLL)
        def _zero_vec(i):
            dst_buf[drow, pl.ds(i * LANES, LANES)] = z16

    out_dma = [None, None]
    for tr in range(N_BLKS):
        bi, bo = tr % 3, tr % 2
        in_dma[bi].wait()
        if out_dma[bo] is not None:
            out_dma[bo].wait()
        cur, prev, obuf = in_bufs[bi], in_bufs[(tr - 1) % 3], out_bufs[bo]
        n_out = min(BLK, OUT_W - tr * BLK)
        if tr == 0:
            zero_row(obuf, 0)
        else:
            map_row(prev, BLK - 1, obuf, 0)
        n_flat = (n_out - 1) * N_VEC

        @plsc.parallel_loop(0, n_flat, 1, unroll=UNROLL)
        def _map_block(i):
            s = lax.shift_right_logical(i, 5)
            off = lax.shift_left(jnp.bitwise_and(i, N_VEC - 1), 4)
            tag16 = cur[s, pl.ds(off, LANES)]
            obuf[s + 1, pl.ds(off, LANES)] = plsc.load_gather(tab_v, [tag16])
        if tr + 2 < N_BLKS:
            in_dma[(tr + 2) % 3] = start_in(tr + 2)
        out_dma[bo] = pltpu.async_copy(
            obuf.at[pl.ds(0, n_out)],
            out_hbm.at[pl.ds(tr * BLK, n_out), pl.ds(c0, COLS_PER_W)],
            out_sems[bo])
    for b in range(2):
        out_dma[b].wait()


def kernel(tags, label_table):
    table_pad = jnp.zeros((TAB_PAD,), jnp.int32).at[:NUM_TAGS].set(label_table)
    out_t = _sc_lookup_t(tags.T, table_pad)
    return out_t.T


# speedup vs baseline: 1.9470x; 1.0302x over previous
import functools

import jax
import jax.numpy as jnp
from jax import lax
from jax.experimental import pallas as pl
from jax.experimental.pallas import tpu as pltpu
from jax.experimental.pallas import tpu_sc as plsc

NUM_TAGS = 44
B, L = 16384, 108
OUT_W = L + 1
LANES = 16
NC, NS = 2, 16
NW = NC * NS
COLS_PER_W = B // NW
BLK = 8
N_BLKS = (L + BLK - 1) // BLK
TAB_PAD = 48
UNROLL = 8
N_VEC = COLS_PER_W // LANES

_MESH = plsc.VectorSubcoreMesh(core_axis_name="c", subcore_axis_name="s")


@functools.partial(
    pl.kernel,
    out_type=jax.ShapeDtypeStruct((OUT_W, B), jnp.int32),
    mesh=_MESH,
    compiler_params=pltpu.CompilerParams(needs_layout_passes=False),
    scratch_types=[
        pltpu.VMEM((TAB_PAD,), jnp.int32),
        pltpu.VMEM((BLK, COLS_PER_W), jnp.int32),
        pltpu.VMEM((BLK, COLS_PER_W), jnp.int32),
        pltpu.VMEM((BLK, COLS_PER_W), jnp.int32),
        pltpu.VMEM((BLK, COLS_PER_W), jnp.int32),
        pltpu.VMEM((BLK, COLS_PER_W), jnp.int32),
        pltpu.SemaphoreType.DMA,
        pltpu.SemaphoreType.DMA,
        pltpu.SemaphoreType.DMA,
        pltpu.SemaphoreType.DMA,
        pltpu.SemaphoreType.DMA,
    ],
)
def _sc_lookup_t(tags_hbm, table_hbm, out_hbm,
                 tab_v, in0, in1, in2, out0, out1,
                 in_sem0, in_sem1, in_sem2, out_sem0, out_sem1):
    wid = lax.axis_index("s") * NC + lax.axis_index("c")
    c0 = wid * COLS_PER_W
    in_bufs, out_bufs = [in0, in1, in2], [out0, out1]
    in_sems, out_sems = [in_sem0, in_sem1, in_sem2], [out_sem0, out_sem1]

    def rows_of(tr):
        return min(BLK, L - tr * BLK)

    def start_in(tr):
        nr = rows_of(tr)
        return pltpu.async_copy(
            tags_hbm.at[pl.ds(tr * BLK, nr), pl.ds(c0, COLS_PER_W)],
            in_bufs[tr % 3].at[pl.ds(0, nr)],
            in_sems[tr % 3])

    in_dma = [start_in(0), start_in(1), None]
    pltpu.sync_copy(table_hbm, tab_v)

    def map_row(src_buf, srow, dst_buf, drow):
        @plsc.parallel_loop(0, N_VEC, 1, unroll=UNROLL)
        def _map_vec(i):
            tag16 = src_buf[srow, pl.ds(i * LANES, LANES)]
            dst_buf[drow, pl.ds(i * LANES, LANES)] = (
                plsc.load_gather(tab_v, [tag16]))

    def zero_row(dst_buf, drow):
        z16 = jnp.zeros((LANES,), jnp.int32)

        @plsc.parallel_loop(0, N_VEC, 1, unroll=UNROLL)
        def _zero_vec(i):
            dst_buf[drow, pl.ds(i * LANES, LANES)] = z16

    out_dma = [None, None]
    for tr in range(N_BLKS):
        bi, bo = tr % 3, tr % 2
        in_dma[bi].wait()
        if out_dma[bo] is not None:
            out_dma[bo].wait()
        cur, prev, obuf = in_bufs[bi], in_bufs[(tr - 1) % 3], out_bufs[bo]
        n_out = min(BLK, OUT_W - tr * BLK)
        if tr == 0:
            zero_row(obuf, 0)
        else:
            map_row(prev, BLK - 1, obuf, 0)
        n_flat = (n_out - 1) * N_VEC

        @plsc.parallel_loop(0, n_flat, 1, unroll=UNROLL)
        def _map_block(i):
            s = lax.shift_right_logical(i, 5)
            off = lax.shift_left(jnp.bitwise_and(i, N_VEC - 1), 4)
            tag16 = cur[s, pl.ds(off, LANES)]
            obuf[s + 1, pl.ds(off, LANES)] = plsc.load_gather(tab_v, [tag16])
        if tr + 2 < N_BLKS:
            in_dma[(tr + 2) % 3] = start_in(tr + 2)
        out_dma[bo] = pltpu.async_copy(
            obuf.at[pl.ds(0, n_out)],
            out_hbm.at[pl.ds(tr * BLK, n_out), pl.ds(c0, COLS_PER_W)],
            out_sems[bo])
    for b in range(2):
        out_dma[b].wait()


def kernel(tags, label_table):
    table_pad = jnp.zeros((TAB_PAD,), jnp.int32).at[:NUM_TAGS].set(label_table)
    out_t = _sc_lookup_t(tags.T, table_pad)
    return out_t.T
